# Initial kernel scaffold; baseline (speedup 1.0000x reference)
import functools

import jax
import jax.numpy as jnp
from jax import lax
from jax.experimental import pallas as pl
from jax.experimental.pallas import tpu as pltpu
from jax.experimental.pallas import tpu_sc as plsc

B = 4
N = 8192
S = 1024
K = 32
R2 = 0.25
CIN = 16
COUT = 64

NSUB = 64
NLANE = 128

NC = 2
NS = 16
NW = NC * NS
ROWS = B * S
RPW = ROWS // NW


def _fps_body(xyz_ref, idx_ref, cen_ref):
    X = xyz_ref[:, 0]
    Y = xyz_ref[:, 1]
    Z = xyz_ref[:, 2]
    ii = (lax.broadcasted_iota(jnp.int32, (NSUB, NLANE), 0) * NLANE
          + lax.broadcasted_iota(jnp.int32, (NSUB, NLANE), 1))
    cen_ref[...] = jnp.zeros((B, S, 8), jnp.float32)

    def body(i, carry):
        dist, far = carry
        idx_ref[:, pl.ds(i, 1)] = far[:, 0, :]
        sel = ii[None] == far
        cx = jnp.sum(jnp.where(sel, X, 0.0), axis=(1, 2), keepdims=True)
        cy = jnp.sum(jnp.where(sel, Y, 0.0), axis=(1, 2), keepdims=True)
        cz = jnp.sum(jnp.where(sel, Z, 0.0), axis=(1, 2), keepdims=True)
        cen_ref[:, pl.ds(i, 1), 0] = cx[:, 0, :]
        cen_ref[:, pl.ds(i, 1), 1] = cy[:, 0, :]
        cen_ref[:, pl.ds(i, 1), 2] = cz[:, 0, :]
        dx = X - cx
        d = dx * dx
        dy = Y - cy
        d = d + dy * dy
        dz = Z - cz
        d = d + dz * dz
        dist = jnp.minimum(dist, d)
        m = jnp.max(dist, axis=(1, 2), keepdims=True)
        far2 = jnp.min(jnp.where(dist == m, ii[None], N), axis=(1, 2),
                       keepdims=True).astype(jnp.int32)
        return dist, far2

    dist0 = jnp.full((B, NSUB, NLANE), 1e10, jnp.float32)
    far0 = jnp.zeros((B, 1, 1), jnp.int32)
    lax.fori_loop(0, S, body, (dist0, far0))


def _fps_call(xyz_f):
    return pl.pallas_call(
        _fps_body,
        out_shape=(
            jax.ShapeDtypeStruct((B, S), jnp.int32),
            jax.ShapeDtypeStruct((B, S, 8), jnp.float32),
        ),
    )(xyz_f)


SBLK = 128


def _compose_weights(W1_ref, W2_ref, W3_ref, b1_ref, b2_ref, b3_ref):
    W21 = lax.dot_general(W2_ref[...], W1_ref[...], (((1,), (0,)), ((), ())),
                          preferred_element_type=jnp.float32)
    We = lax.dot_general(W3_ref[...], W21, (((1,), (0,)), ((), ())),
                         preferred_element_type=jnp.float32)
    bmid = lax.dot_general(b1_ref[...], W2_ref[...], (((1,), (1,)), ((), ())),
                           preferred_element_type=jnp.float32) + b2_ref[...]
    ber = lax.dot_general(bmid, W3_ref[...], (((1,), (1,)), ((), ())),
                          preferred_element_type=jnp.float32) + b3_ref[...]
    return We, ber


def _distw_body(cen_ref, xyzp_ref, W1_ref, W2_ref, W3_ref, b1_ref, b2_ref,
                b3_ref, dist_ref, w_ref):
    a = cen_ref[0]
    xp = xyzp_ref[0]
    mm = lax.dot_general(a, xp, (((1,), (0,)), ((), ())),
                         preferred_element_type=jnp.float32)
    asq = jnp.sum(a * a, axis=1, keepdims=True)
    sq = xp * xp
    bsq = sq[0:1] + sq[1:2] + sq[2:3]
    d = (-2.0) * mm
    d = d + asq
    d = d + bsq
    dist_ref[0] = d

    We, ber = _compose_weights(W1_ref, W2_ref, W3_ref, b1_ref, b2_ref, b3_ref)
    Wxp = jnp.concatenate([We[:, 0:3], jnp.zeros((COUT, 5), jnp.float32)],
                          axis=1)
    wc = lax.dot_general(a, Wxp, (((1,), (1,)), ((), ())),
                         preferred_element_type=jnp.float32)
    w_ref[0] = ber - wc


def _distw_call(cen, xyz_p, W1, W2, W3, b1r, b2r, b3r):
    grid = (B, S // SBLK)
    return pl.pallas_call(
        _distw_body,
        grid=grid,
        in_specs=[
            pl.BlockSpec((1, SBLK, 8), lambda b, j: (b, j, 0)),
            pl.BlockSpec((1, 8, N), lambda b, j: (b, 0, 0)),
            pl.BlockSpec((32, 16), lambda b, j: (0, 0)),
            pl.BlockSpec((32, 32), lambda b, j: (0, 0)),
            pl.BlockSpec((64, 32), lambda b, j: (0, 0)),
            pl.BlockSpec((1, 32), lambda b, j: (0, 0)),
            pl.BlockSpec((1, 32), lambda b, j: (0, 0)),
            pl.BlockSpec((1, 64), lambda b, j: (0, 0)),
        ],
        out_specs=[
            pl.BlockSpec((1, SBLK, N), lambda b, j: (b, j, 0)),
            pl.BlockSpec((1, SBLK, 64), lambda b, j: (b, j, 0)),
        ],
        out_shape=(
            jax.ShapeDtypeStruct((B, S, N), jnp.float32),
            jax.ShapeDtypeStruct((B, S, 64), jnp.float32),
        ),
    )(cen, xyz_p, W1, W2, W3, b1r, b2r, b3r)


def _u_body(g_ref, W1_ref, W2_ref, W3_ref, b1_ref, b2_ref, b3_ref, u_ref):
    We, _ = _compose_weights(W1_ref, W2_ref, W3_ref, b1_ref, b2_ref, b3_ref)
    u_ref[0] = lax.dot_general(g_ref[0], We, (((1,), (1,)), ((), ())),
                               preferred_element_type=jnp.float32)


def _u_call(g16, W1, W2, W3, b1r, b2r, b3r):
    return pl.pallas_call(
        _u_body,
        grid=(B,),
        in_specs=[
            pl.BlockSpec((1, N, CIN), lambda b: (b, 0, 0)),
            pl.BlockSpec((32, 16), lambda b: (0, 0)),
            pl.BlockSpec((32, 32), lambda b: (0, 0)),
            pl.BlockSpec((64, 32), lambda b: (0, 0)),
            pl.BlockSpec((1, 32), lambda b: (0, 0)),
            pl.BlockSpec((1, 32), lambda b: (0, 0)),
            pl.BlockSpec((1, 64), lambda b: (0, 0)),
        ],
        out_specs=pl.BlockSpec((1, N, COUT), lambda b: (b, 0, 0)),
        out_shape=jax.ShapeDtypeStruct((B, N, COUT), jnp.float32),
    )(g16, W1, W2, W3, b1r, b2r, b3r)


def _sc_body(dist_hbm, u_hbm, w_hbm, out_hbm, dbuf, selbuf, gidx, urows,
             wrow, obuf, sem):
    wid = lax.axis_index("s") * NC + lax.axis_index("c")
    iota = lax.iota(jnp.int32, 16)

    def per_row(t, _):
        row = wid * RPW + t
        b = row // S
        nbase = b * N
        pltpu.sync_copy(dist_hbm.at[row], dbuf)

        def cond(c):
            st, off = c
            return jnp.logical_and(st < N // 16, off < K)

        def sbody(c):
            st, off = c
            d = dbuf[pl.ds(st * 16, 16)]
            m = d <= R2
            ids = iota + (st * 16 + nbase)
            plsc.store_compressed(selbuf.at[pl.ds(off, 16)], ids, m)
            cnt = jnp.max(plsc.all_reduce_population_count(m))
            return st + 1, off + cnt

        _, off = lax.while_loop(cond, sbody, (0, 0))
        offm = jnp.maximum(off - 1, 0)
        gidx[pl.ds(0, 16)] = plsc.load_gather(selbuf, [jnp.minimum(iota, offm)])
        gidx[pl.ds(16, 16)] = plsc.load_gather(
            selbuf, [jnp.minimum(iota + 16, offm)])

        pltpu.async_copy(u_hbm.at[gidx], urows, sem).wait()
        pltpu.sync_copy(w_hbm.at[row], wrow)

        acc = [urows[0, pl.ds(16 * j, 16)] for j in range(4)]
        for k in range(1, K):
            for j in range(4):
                acc[j] = jnp.maximum(acc[j], urows[k, pl.ds(16 * j, 16)])
        for j in range(4):
            obuf[pl.ds(16 * j, 16)] = acc[j] + wrow[pl.ds(16 * j, 16)]
        pltpu.sync_copy(obuf, out_hbm.at[row])
        return 0

    lax.fori_loop(0, RPW, per_row, 0)


_sc_call = functools.partial(
    pl.kernel,
    out_type=jax.ShapeDtypeStruct((ROWS, COUT), jnp.float32),
    mesh=plsc.VectorSubcoreMesh(core_axis_name="c", subcore_axis_name="s"),
    scratch_types=[
        pltpu.VMEM((N,), jnp.float32),
        pltpu.VMEM((48,), jnp.int32),
        pltpu.VMEM((K,), jnp.int32),
        pltpu.VMEM((K, COUT), jnp.float32),
        pltpu.VMEM((COUT,), jnp.float32),
        pltpu.VMEM((COUT,), jnp.float32),
        pltpu.SemaphoreType.DMA,
    ],
)(_sc_body)


def kernel(xyz, features, W1, b1, W2, b2, W3, b3):
    xyz_t = jnp.transpose(xyz, (0, 2, 1))
    xyz_f = xyz_t.reshape(B, 3, NSUB, NLANE)
    xyz_p = jnp.concatenate(
        [xyz_t, jnp.zeros((B, 5, N), jnp.float32)], axis=1)
    g16 = jnp.concatenate([xyz, features], axis=-1)
    b1r = b1.reshape(1, 32)
    b2r = b2.reshape(1, 32)
    b3r = b3.reshape(1, 64)

    fps_idx, cen = _fps_call(xyz_f)
    dist, w = _distw_call(cen, xyz_p, W1, W2, W3, b1r, b2r, b3r)
    u = _u_call(g16, W1, W2, W3, b1r, b2r, b3r)

    nf = _sc_call(dist.reshape(ROWS, N), u.reshape(B * N, COUT),
                  w.reshape(ROWS, COUT))

    new_xyz = cen[:, :, 0:3]
    new_features = jnp.transpose(
        nf.reshape(B, S, COUT), (0, 2, 1))[:, :, None, :]
    return new_xyz, new_features, fps_idx


# speedup vs baseline: 12.1224x; 12.1224x over previous
import functools

import jax
import jax.numpy as jnp
from jax import lax
from jax.experimental import pallas as pl
from jax.experimental.pallas import tpu as pltpu
from jax.experimental.pallas import tpu_sc as plsc

B = 4
N = 8192
S = 1024
K = 32
R2 = 0.25
CIN = 16
COUT = 64

NSUB = 64
NLANE = 128

NC = 2
NS = 16
NW = NC * NS
ROWS = B * S
RPW = ROWS // NW


def _fps_body(xyz_ref, idx_ref, cen_ref):
    X = xyz_ref[:, 0]
    Y = xyz_ref[:, 1]
    Z = xyz_ref[:, 2]
    ii = (lax.broadcasted_iota(jnp.int32, (NSUB, NLANE), 0) * NLANE
          + lax.broadcasted_iota(jnp.int32, (NSUB, NLANE), 1))
    si = lax.broadcasted_iota(jnp.int32, (1, S), 1)

    def body(i, carry):
        dist, far, idxs, cxs, cys, czs = carry
        lane = si == i
        idxs = jnp.where(lane, far[:, :, 0], idxs)
        sel = ii[None] == far
        cx = jnp.sum(jnp.where(sel, X, 0.0), axis=(1, 2), keepdims=True)
        cy = jnp.sum(jnp.where(sel, Y, 0.0), axis=(1, 2), keepdims=True)
        cz = jnp.sum(jnp.where(sel, Z, 0.0), axis=(1, 2), keepdims=True)
        cxs = jnp.where(lane, cx[:, :, 0], cxs)
        cys = jnp.where(lane, cy[:, :, 0], cys)
        czs = jnp.where(lane, cz[:, :, 0], czs)
        dx = X - cx
        d = dx * dx
        dy = Y - cy
        d = d + dy * dy
        dz = Z - cz
        d = d + dz * dz
        dist = jnp.minimum(dist, d)
        m = jnp.max(dist, axis=(1, 2), keepdims=True)
        far2 = jnp.min(jnp.where(dist == m, ii[None], N), axis=(1, 2),
                       keepdims=True).astype(jnp.int32)
        return dist, far2, idxs, cxs, cys, czs

    dist0 = jnp.full((B, NSUB, NLANE), 1e10, jnp.float32)
    far0 = jnp.zeros((B, 1, 1), jnp.int32)
    z = jnp.zeros((B, S), jnp.float32)
    iz = jnp.zeros((B, S), jnp.int32)
    _, _, idxs, cxs, cys, czs = lax.fori_loop(
        0, S, body, (dist0, far0, iz, z, z, z))
    idx_ref[...] = idxs
    cen_ref[...] = jnp.zeros((B, 8, S), jnp.float32)
    cen_ref[:, 0, :] = cxs
    cen_ref[:, 1, :] = cys
    cen_ref[:, 2, :] = czs


def _fps_call(xyz_f):
    return pl.pallas_call(
        _fps_body,
        out_shape=(
            jax.ShapeDtypeStruct((B, S), jnp.int32),
            jax.ShapeDtypeStruct((B, 8, S), jnp.float32),
        ),
    )(xyz_f)


SBLK = 128


def _compose_weights(W1_ref, W2_ref, W3_ref, b1_ref, b2_ref, b3_ref):
    W21 = lax.dot_general(W2_ref[...], W1_ref[...], (((1,), (0,)), ((), ())),
                          preferred_element_type=jnp.float32)
    We = lax.dot_general(W3_ref[...], W21, (((1,), (0,)), ((), ())),
                         preferred_element_type=jnp.float32)
    bmid = lax.dot_general(b1_ref[...], W2_ref[...], (((1,), (1,)), ((), ())),
                           preferred_element_type=jnp.float32) + b2_ref[...]
    ber = lax.dot_general(bmid, W3_ref[...], (((1,), (1,)), ((), ())),
                          preferred_element_type=jnp.float32) + b3_ref[...]
    return We, ber


def _distw_body(cen_ref, xyzp_ref, W1_ref, W2_ref, W3_ref, b1_ref, b2_ref,
                b3_ref, dist_ref, w_ref):
    a = cen_ref[0]
    xp = xyzp_ref[0]
    mm = lax.dot_general(a, xp, (((1,), (0,)), ((), ())),
                         preferred_element_type=jnp.float32)
    asq = jnp.sum(a * a, axis=1, keepdims=True)
    sq = xp * xp
    bsq = sq[0:1] + sq[1:2] + sq[2:3]
    d = (-2.0) * mm
    d = d + asq
    d = d + bsq
    dist_ref[0] = d

    We, ber = _compose_weights(W1_ref, W2_ref, W3_ref, b1_ref, b2_ref, b3_ref)
    Wxp = jnp.concatenate([We[:, 0:3], jnp.zeros((COUT, 5), jnp.float32)],
                          axis=1)
    wc = lax.dot_general(a, Wxp, (((1,), (1,)), ((), ())),
                         preferred_element_type=jnp.float32)
    w_ref[0] = ber - wc


def _distw_call(cen, xyz_p, W1, W2, W3, b1r, b2r, b3r):
    grid = (B, S // SBLK)
    return pl.pallas_call(
        _distw_body,
        grid=grid,
        in_specs=[
            pl.BlockSpec((1, SBLK, 8), lambda b, j: (b, j, 0)),
            pl.BlockSpec((1, 8, N), lambda b, j: (b, 0, 0)),
            pl.BlockSpec((32, 16), lambda b, j: (0, 0)),
            pl.BlockSpec((32, 32), lambda b, j: (0, 0)),
            pl.BlockSpec((64, 32), lambda b, j: (0, 0)),
            pl.BlockSpec((1, 32), lambda b, j: (0, 0)),
            pl.BlockSpec((1, 32), lambda b, j: (0, 0)),
            pl.BlockSpec((1, 64), lambda b, j: (0, 0)),
        ],
        out_specs=[
            pl.BlockSpec((1, SBLK, N), lambda b, j: (b, j, 0)),
            pl.BlockSpec((1, SBLK, 64), lambda b, j: (b, j, 0)),
        ],
        out_shape=(
            jax.ShapeDtypeStruct((B, S, N), jnp.float32),
            jax.ShapeDtypeStruct((B, S, 64), jnp.float32),
        ),
    )(cen, xyz_p, W1, W2, W3, b1r, b2r, b3r)


def _u_body(g_ref, W1_ref, W2_ref, W3_ref, b1_ref, b2_ref, b3_ref, u_ref):
    We, _ = _compose_weights(W1_ref, W2_ref, W3_ref, b1_ref, b2_ref, b3_ref)
    u_ref[0] = lax.dot_general(g_ref[0], We, (((1,), (1,)), ((), ())),
                               preferred_element_type=jnp.float32)


def _u_call(g16, W1, W2, W3, b1r, b2r, b3r):
    return pl.pallas_call(
        _u_body,
        grid=(B,),
        in_specs=[
            pl.BlockSpec((1, N, CIN), lambda b: (b, 0, 0)),
            pl.BlockSpec((32, 16), lambda b: (0, 0)),
            pl.BlockSpec((32, 32), lambda b: (0, 0)),
            pl.BlockSpec((64, 32), lambda b: (0, 0)),
            pl.BlockSpec((1, 32), lambda b: (0, 0)),
            pl.BlockSpec((1, 32), lambda b: (0, 0)),
            pl.BlockSpec((1, 64), lambda b: (0, 0)),
        ],
        out_specs=pl.BlockSpec((1, N, COUT), lambda b: (b, 0, 0)),
        out_shape=jax.ShapeDtypeStruct((B, N, COUT), jnp.float32),
    )(g16, W1, W2, W3, b1r, b2r, b3r)


def _sc_body(dist_hbm, u_hbm, w_hbm, out_hbm, dbuf, selbuf, gidx, urows,
             wrow, obuf, sem):
    wid = lax.axis_index("s") * NC + lax.axis_index("c")
    iota = lax.iota(jnp.int32, 16)

    def per_row(t, _):
        row = wid * RPW + t
        b = row // S
        nbase = b * N
        pltpu.sync_copy(dist_hbm.at[row], dbuf)

        def sbody(st, off):
            d = dbuf[pl.ds(st * 16, 16)]
            m = jnp.logical_and(d <= R2, off < K)
            ids = iota + (st * 16 + nbase)
            csum = plsc.cumsum(m.astype(jnp.int32))
            plsc.store_scatter(selbuf, [off + csum - 1], ids, mask=m)
            return off + csum[15]

        off = lax.fori_loop(0, N // 16, sbody, 0)
        offm = jnp.maximum(off - 1, 0)
        gidx[pl.ds(0, 16)] = plsc.load_gather(selbuf, [jnp.minimum(iota, offm)])
        gidx[pl.ds(16, 16)] = plsc.load_gather(
            selbuf, [jnp.minimum(iota + 16, offm)])

        pltpu.async_copy(u_hbm.at[gidx], urows, sem).wait()
        pltpu.sync_copy(w_hbm.at[row], wrow)

        acc = [urows[0, pl.ds(16 * j, 16)] for j in range(4)]
        for k in range(1, K):
            for j in range(4):
                acc[j] = jnp.maximum(acc[j], urows[k, pl.ds(16 * j, 16)])
        for j in range(4):
            obuf[pl.ds(16 * j, 16)] = acc[j] + wrow[pl.ds(16 * j, 16)]
        pltpu.sync_copy(obuf, out_hbm.at[row])
        return 0

    lax.fori_loop(0, RPW, per_row, 0)


@functools.cache
def _sc_call():
    return pl.kernel(
        _sc_body,
        out_type=jax.ShapeDtypeStruct((ROWS, COUT), jnp.float32),
        compiler_params=pltpu.CompilerParams(
            needs_layout_passes=False, use_tc_tiling_on_sc=False),
        mesh=plsc.VectorSubcoreMesh(core_axis_name="c", subcore_axis_name="s"),
        scratch_types=[
            pltpu.VMEM((N,), jnp.float32),
            pltpu.VMEM((64,), jnp.int32),
            pltpu.VMEM((K,), jnp.int32),
            pltpu.VMEM((K, COUT), jnp.float32),
            pltpu.VMEM((COUT,), jnp.float32),
            pltpu.VMEM((COUT,), jnp.float32),
            pltpu.SemaphoreType.DMA,
        ],
    )


def kernel(xyz, features, W1, b1, W2, b2, W3, b3):
    xyz_t = jnp.transpose(xyz, (0, 2, 1))
    xyz_f = xyz_t.reshape(B, 3, NSUB, NLANE)
    xyz_p = jnp.concatenate(
        [xyz_t, jnp.zeros((B, 5, N), jnp.float32)], axis=1)
    g16 = jnp.concatenate([xyz, features], axis=-1)
    b1r = b1.reshape(1, 32)
    b2r = b2.reshape(1, 32)
    b3r = b3.reshape(1, 64)

    fps_idx, cen = _fps_call(xyz_f)
    cen_rm = jnp.transpose(cen, (0, 2, 1))
    dist, w = _distw_call(cen_rm, xyz_p, W1, W2, W3, b1r, b2r, b3r)
    u = _u_call(g16, W1, W2, W3, b1r, b2r, b3r)

    nf = _sc_call()(dist.reshape(ROWS, N), u.reshape(B * N, COUT),
                    w.reshape(ROWS, COUT))

    new_xyz = cen_rm[:, :, 0:3]
    new_features = jnp.transpose(
        nf.reshape(B, S, COUT), (0, 2, 1))[:, :, None, :]
    return new_xyz, new_features, fps_idx


# speedup vs baseline: 20.5275x; 1.6933x over previous
import functools

import jax
import jax.numpy as jnp
from jax import lax
from jax.experimental import pallas as pl
from jax.experimental.pallas import tpu as pltpu
from jax.experimental.pallas import tpu_sc as plsc

B = 4
N = 8192
S = 1024
K = 32
R2 = 0.25
CIN = 16
COUT = 64

NSUB = 64
NLANE = 128

NC = 2
NS = 16
NW = NC * NS
ROWS = B * S
RPW = ROWS // NW


def _fps_body(xyz_ref, idx_ref, cen_ref):
    X = xyz_ref[:, 0]
    Y = xyz_ref[:, 1]
    Z = xyz_ref[:, 2]
    ii = (lax.broadcasted_iota(jnp.int32, (NSUB, NLANE), 0) * NLANE
          + lax.broadcasted_iota(jnp.int32, (NSUB, NLANE), 1))
    si = lax.broadcasted_iota(jnp.int32, (1, S), 1)

    def body(i, carry):
        dist, far, idxs, cxs, cys, czs = carry
        lane = si == i
        idxs = jnp.where(lane, far[:, :, 0], idxs)
        sel = ii[None] == far
        cx = jnp.sum(jnp.where(sel, X, 0.0), axis=(1, 2), keepdims=True)
        cy = jnp.sum(jnp.where(sel, Y, 0.0), axis=(1, 2), keepdims=True)
        cz = jnp.sum(jnp.where(sel, Z, 0.0), axis=(1, 2), keepdims=True)
        cxs = jnp.where(lane, cx[:, :, 0], cxs)
        cys = jnp.where(lane, cy[:, :, 0], cys)
        czs = jnp.where(lane, cz[:, :, 0], czs)
        dx = X - cx
        d = dx * dx
        dy = Y - cy
        d = d + dy * dy
        dz = Z - cz
        d = d + dz * dz
        dist = jnp.minimum(dist, d)
        m = jnp.max(dist, axis=(1, 2), keepdims=True)
        far2 = jnp.min(jnp.where(dist == m, ii[None], N), axis=(1, 2),
                       keepdims=True).astype(jnp.int32)
        return dist, far2, idxs, cxs, cys, czs

    dist0 = jnp.full((B, NSUB, NLANE), 1e10, jnp.float32)
    far0 = jnp.zeros((B, 1, 1), jnp.int32)
    z = jnp.zeros((B, S), jnp.float32)
    iz = jnp.zeros((B, S), jnp.int32)
    _, _, idxs, cxs, cys, czs = lax.fori_loop(
        0, S, body, (dist0, far0, iz, z, z, z))
    idx_ref[...] = idxs
    cen_ref[...] = jnp.zeros((B, 8, S), jnp.float32)
    cen_ref[:, 0, :] = cxs
    cen_ref[:, 1, :] = cys
    cen_ref[:, 2, :] = czs


def _fps_call(xyz_f):
    return pl.pallas_call(
        _fps_body,
        out_shape=(
            jax.ShapeDtypeStruct((B, S), jnp.int32),
            jax.ShapeDtypeStruct((B, 8, S), jnp.float32),
        ),
    )(xyz_f)


SBLK = 128


def _compose_weights(W1_ref, W2_ref, W3_ref, b1_ref, b2_ref, b3_ref):
    W21 = lax.dot_general(W2_ref[...], W1_ref[...], (((1,), (0,)), ((), ())),
                          preferred_element_type=jnp.float32)
    We = lax.dot_general(W3_ref[...], W21, (((1,), (0,)), ((), ())),
                         preferred_element_type=jnp.float32)
    bmid = lax.dot_general(b1_ref[...], W2_ref[...], (((1,), (1,)), ((), ())),
                           preferred_element_type=jnp.float32) + b2_ref[...]
    ber = lax.dot_general(bmid, W3_ref[...], (((1,), (1,)), ((), ())),
                          preferred_element_type=jnp.float32) + b3_ref[...]
    return We, ber


CHUNK = 512
NCH = N // CHUNK


def _distw_body(cen_ref, xyzp_ref, W1_ref, W2_ref, W3_ref, b1_ref, b2_ref,
                b3_ref, dist_ref, w_ref, cnt_ref):
    a = cen_ref[...]
    xp = xyzp_ref[0]
    mm = lax.dot_general(a, xp, (((1,), (0,)), ((), ())),
                         preferred_element_type=jnp.float32)
    asq = jnp.sum(a * a, axis=1, keepdims=True)
    sq = xp * xp
    bsq = sq[0:1] + sq[1:2] + sq[2:3]
    d = (-2.0) * mm
    d = d + asq
    d = d + bsq
    dist_ref[...] = d

    msk = jnp.where(d <= R2, 1.0, 0.0).reshape(SBLK, NCH, CHUNK)
    cnt_ref[...] = jnp.sum(msk, axis=2).astype(jnp.int32)

    We, ber = _compose_weights(W1_ref, W2_ref, W3_ref, b1_ref, b2_ref, b3_ref)
    Wxp = jnp.concatenate([We[:, 0:3], jnp.zeros((COUT, 5), jnp.float32)],
                          axis=1)
    wc = lax.dot_general(a, Wxp, (((1,), (1,)), ((), ())),
                         preferred_element_type=jnp.float32)
    w_ref[...] = ber - wc


def _distw_call(cen_rm2, xyz_p, W1, W2, W3, b1r, b2r, b3r):
    grid = (B, S // SBLK)
    nj = S // SBLK
    return pl.pallas_call(
        _distw_body,
        grid=grid,
        in_specs=[
            pl.BlockSpec((SBLK, 8), lambda b, j, nj=nj: (b * nj + j, 0)),
            pl.BlockSpec((1, 8, N), lambda b, j: (b, 0, 0)),
            pl.BlockSpec((32, 16), lambda b, j: (0, 0)),
            pl.BlockSpec((32, 32), lambda b, j: (0, 0)),
            pl.BlockSpec((64, 32), lambda b, j: (0, 0)),
            pl.BlockSpec((1, 32), lambda b, j: (0, 0)),
            pl.BlockSpec((1, 32), lambda b, j: (0, 0)),
            pl.BlockSpec((1, 64), lambda b, j: (0, 0)),
        ],
        out_specs=[
            pl.BlockSpec((SBLK, N), lambda b, j, nj=nj: (b * nj + j, 0)),
            pl.BlockSpec((SBLK, 64), lambda b, j, nj=nj: (b * nj + j, 0)),
            pl.BlockSpec((SBLK, NCH), lambda b, j, nj=nj: (b * nj + j, 0)),
        ],
        out_shape=(
            jax.ShapeDtypeStruct((ROWS, N), jnp.float32),
            jax.ShapeDtypeStruct((ROWS, 64), jnp.float32),
            jax.ShapeDtypeStruct((ROWS, NCH), jnp.int32),
        ),
    )(cen_rm2, xyz_p, W1, W2, W3, b1r, b2r, b3r)


def _u_body(g_ref, W1_ref, W2_ref, W3_ref, b1_ref, b2_ref, b3_ref, u_ref):
    We, _ = _compose_weights(W1_ref, W2_ref, W3_ref, b1_ref, b2_ref, b3_ref)
    u_ref[...] = lax.dot_general(g_ref[...], We, (((1,), (1,)), ((), ())),
                                 preferred_element_type=jnp.float32)


def _u_call(g16f, W1, W2, W3, b1r, b2r, b3r):
    return pl.pallas_call(
        _u_body,
        grid=(B,),
        in_specs=[
            pl.BlockSpec((N, CIN), lambda b: (b, 0)),
            pl.BlockSpec((32, 16), lambda b: (0, 0)),
            pl.BlockSpec((32, 32), lambda b: (0, 0)),
            pl.BlockSpec((64, 32), lambda b: (0, 0)),
            pl.BlockSpec((1, 32), lambda b: (0, 0)),
            pl.BlockSpec((1, 32), lambda b: (0, 0)),
            pl.BlockSpec((1, 64), lambda b: (0, 0)),
        ],
        out_specs=pl.BlockSpec((N, COUT), lambda b: (b, 0)),
        out_shape=jax.ShapeDtypeStruct((B * N, COUT), jnp.float32),
    )(g16f, W1, W2, W3, b1r, b2r, b3r)


def _sc_body(dist_hbm, u_hbm, w_hbm, cnt_hbm, out_hbm, dbufA, dbufB, cbuf,
             wbuf, obuf, selbuf, clist, gidx, urows, semA, semB, semG):
    wid = lax.axis_index("s") * NC + lax.axis_index("c")
    iota = lax.iota(jnp.int32, 16)
    base_row = wid * RPW
    b = base_row // S
    nbase = b * N

    pltpu.sync_copy(cnt_hbm.at[pl.ds(base_row, RPW)], cbuf)
    pltpu.sync_copy(w_hbm.at[pl.ds(base_row, RPW)], wbuf)
    cpA = pltpu.async_copy(dist_hbm.at[base_row], dbufA, semA)

    def process(t, dbuf):
        row = base_row + t
        counts = cbuf[t, pl.ds(0, NCH)]
        csum = plsc.cumsum(counts)
        excl = csum - counts
        pm = jnp.logical_and(counts > 0, excl < K)
        pos = plsc.cumsum(pm.astype(jnp.int32)) - 1
        plsc.store_scatter(clist, [pos], iota, mask=pm)
        ntr = jnp.sum(pm.astype(jnp.int32))
        selbuf[pl.ds(0, 16)] = jnp.broadcast_to(nbase + N - 1, (16,))

        def chunk_body(tt, off):
            c = plsc.load_gather(clist, [jnp.broadcast_to(tt, (16,))])[0]
            cb = c * CHUNK
            for s in range(CHUNK // 16):
                d16 = dbuf[pl.ds(cb + s * 16, 16)]
                m = d16 <= R2
                ids = iota + (cb + s * 16 + nbase)
                cs = plsc.cumsum(m.astype(jnp.int32))
                plsc.store_scatter(
                    selbuf, [jnp.minimum(off + cs - 1, 63)], ids, mask=m)
                off = off + cs[15]
            return off

        off = lax.fori_loop(0, ntr, chunk_body, 0)
        offm = jnp.maximum(jnp.minimum(off, K) - 1, 0)
        gidx[pl.ds(0, 16)] = plsc.load_gather(selbuf, [jnp.minimum(iota, offm)])
        gidx[pl.ds(16, 16)] = plsc.load_gather(
            selbuf, [jnp.minimum(iota + 16, offm)])

        pltpu.async_copy(u_hbm.at[gidx], urows, semG).wait()

        acc = [urows[0, pl.ds(16 * j, 16)] for j in range(4)]
        for k in range(1, K):
            for j in range(4):
                acc[j] = jnp.maximum(acc[j], urows[k, pl.ds(16 * j, 16)])
        for j in range(4):
            obuf[t, pl.ds(16 * j, 16)] = acc[j] + wbuf[t, pl.ds(16 * j, 16)]

    def pair_body(p, _):
        t0 = 2 * p
        cpA = pltpu.make_async_copy(dist_hbm.at[base_row], dbufA, semA)
        cpA.wait()
        nxt = jnp.minimum(base_row + t0 + 1, ROWS - 1)
        pltpu.async_copy(dist_hbm.at[nxt], dbufB, semB)
        process(t0, dbufA)
        pltpu.make_async_copy(dist_hbm.at[base_row], dbufB, semB).wait()
        nxt2 = jnp.minimum(base_row + t0 + 2, ROWS - 1)
        pltpu.async_copy(dist_hbm.at[nxt2], dbufA, semA)
        process(t0 + 1, dbufB)
        return 0

    lax.fori_loop(0, RPW // 2, pair_body, 0)
    pltpu.make_async_copy(dist_hbm.at[base_row], dbufA, semA).wait()
    pltpu.sync_copy(obuf, out_hbm.at[pl.ds(base_row, RPW)])


@functools.cache
def _sc_call():
    return pl.kernel(
        _sc_body,
        out_type=jax.ShapeDtypeStruct((ROWS, COUT), jnp.float32),
        compiler_params=pltpu.CompilerParams(
            needs_layout_passes=False, use_tc_tiling_on_sc=False),
        mesh=plsc.VectorSubcoreMesh(core_axis_name="c", subcore_axis_name="s"),
        scratch_types=[
            pltpu.VMEM((N,), jnp.float32),
            pltpu.VMEM((N,), jnp.float32),
            pltpu.VMEM((RPW, NCH), jnp.int32),
            pltpu.VMEM((RPW, COUT), jnp.float32),
            pltpu.VMEM((RPW, COUT), jnp.float32),
            pltpu.VMEM((64,), jnp.int32),
            pltpu.VMEM((16,), jnp.int32),
            pltpu.VMEM((K,), jnp.int32),
            pltpu.VMEM((K, COUT), jnp.float32),
            pltpu.SemaphoreType.DMA,
            pltpu.SemaphoreType.DMA,
            pltpu.SemaphoreType.DMA,
        ],
    )


def kernel(xyz, features, W1, b1, W2, b2, W3, b3):
    xyz_t = jnp.transpose(xyz, (0, 2, 1))
    xyz_f = xyz_t.reshape(B, 3, NSUB, NLANE)
    xyz_p = jnp.concatenate(
        [xyz_t, jnp.zeros((B, 5, N), jnp.float32)], axis=1)
    g16f = jnp.concatenate([xyz, features], axis=-1).reshape(B * N, CIN)
    b1r = b1.reshape(1, 32)
    b2r = b2.reshape(1, 32)
    b3r = b3.reshape(1, 64)

    fps_idx, cen = _fps_call(xyz_f)
    cen_rm = jnp.transpose(cen, (0, 2, 1))
    dist, w, cnts = _distw_call(cen_rm.reshape(ROWS, 8), xyz_p,
                                W1, W2, W3, b1r, b2r, b3r)
    u = _u_call(g16f, W1, W2, W3, b1r, b2r, b3r)

    nf = _sc_call()(dist, u, w, cnts)

    new_xyz = cen_rm[:, :, 0:3]
    new_features = jnp.transpose(
        nf.reshape(B, S, COUT), (0, 2, 1))[:, :, None, :]
    return new_xyz, new_features, fps_idx
